# baseline (speedup 1.0000x reference)
import functools

import jax
import jax.numpy as jnp
from jax import lax
from jax.experimental import pallas as pl
from jax.experimental.pallas import tpu as pltpu
from jax.experimental.pallas import tpu_sc as plsc

NC = 2
NS = 16
NW = NC * NS
CH = 128
NBUF = 2


def _mesh():
    return plsc.VectorSubcoreMesh(
        core_axis_name="c", subcore_axis_name="s", num_cores=NC, num_subcores=NS
    )


def _make_deg(n_pad, e_pad):
    epw = e_pad // NW
    n_chunks = epw // CH
    rows_per_sub = n_pad // NS
    grp = 8

    @functools.partial(
        pl.kernel,
        out_type=jax.ShapeDtypeStruct((2 * n_pad, 128), jnp.float32),
        mesh=_mesh(),
        scratch_types=[
            pltpu.VMEM((n_chunks, CH), jnp.int32),
            pltpu.VMEM((CH, 128), jnp.float32),
            pltpu.VMEM_SHARED((n_pad, 128), jnp.float32),
            pltpu.SemaphoreType.DMA,
        ],
    )
    def deg_kernel(dst_hbm, ones_hbm, z_hbm, out_hbm, didx, ones_v, acc, sem):
        c = lax.axis_index("c")
        s = lax.axis_index("s")
        wid = c * NS + s
        r0 = s * rows_per_sub
        row_slc = pl.ds(r0, rows_per_sub)
        pltpu.sync_copy(z_hbm.at[row_slc], acc.at[row_slc])
        pltpu.sync_copy(ones_hbm, ones_v)
        pltpu.sync_copy(dst_hbm.at[pl.ds(wid * n_chunks, n_chunks)], didx)
        plsc.subcore_barrier()

        @pl.loop(0, n_chunks, step=grp)
        def _(g):
            for j in range(grp):
                pltpu.sync_copy(ones_v, acc.at[didx.at[g + j]], add=True)

        plsc.subcore_barrier()
        pltpu.sync_copy(
            acc.at[row_slc], out_hbm.at[pl.ds(c * n_pad + r0, rows_per_sub)]
        )

    return deg_kernel


def _make_prop(n_pad, e_pad, d):
    epw = e_pad // NW
    n_chunks = epw // CH
    rows_per_sub = n_pad // NS

    @functools.partial(
        pl.kernel,
        out_type=jax.ShapeDtypeStruct((2 * n_pad, d), jnp.float32),
        mesh=_mesh(),
        scratch_types=[
            pltpu.VMEM((CH,), jnp.int32),
            pltpu.VMEM((CH,), jnp.int32),
            pltpu.VMEM((CH, d), jnp.float32),
            pltpu.VMEM_SHARED((n_pad, d), jnp.float32),
        ],
    )
    def prop_kernel(s_hbm, src_hbm, dst_hbm, z_hbm, out_hbm,
                    sidx, didx, rows, acc):
        c = lax.axis_index("c")
        s = lax.axis_index("s")
        wid = c * NS + s
        r0 = s * rows_per_sub
        row_slc = pl.ds(r0, rows_per_sub)
        pltpu.sync_copy(z_hbm.at[row_slc], acc.at[row_slc])
        plsc.subcore_barrier()

        base = wid * epw

        @pl.loop(0, n_chunks)
        def _(t):
            off = base + t * CH
            pltpu.sync_copy(src_hbm.at[pl.ds(off, CH)], sidx)
            pltpu.sync_copy(dst_hbm.at[pl.ds(off, CH)], didx)
            pltpu.sync_copy(s_hbm.at[sidx], rows)
            pltpu.sync_copy(rows, acc.at[didx], add=True)

        plsc.subcore_barrier()
        pltpu.sync_copy(
            acc.at[row_slc], out_hbm.at[pl.ds(c * n_pad + r0, rows_per_sub)]
        )

    return prop_kernel


def _dinv_block(d0, d1):
    deg = d0[:, 0:1] + d1[:, 0:1] + 1.0
    return lax.rsqrt(jnp.maximum(deg, 1.0))


def _tc1_body(x_ref, d0_ref, d1_ref, o_ref):
    o_ref[...] = x_ref[...] * _dinv_block(d0_ref[...], d1_ref[...])


def _tc2_body(a0_ref, a1_ref, s1_ref, d0_ref, d1_ref, w1_ref, b1_ref, w2_ref,
              o_ref):
    dinv = _dinv_block(d0_ref[...], d1_ref[...])
    p1 = (a0_ref[...] + a1_ref[...] + s1_ref[...]) * dinv
    h1 = jax.nn.relu(
        jnp.dot(p1, w1_ref[...], preferred_element_type=jnp.float32)
        + b1_ref[...]
    )
    m2 = jnp.dot(h1, w2_ref[...], preferred_element_type=jnp.float32)
    o_ref[...] = m2 * dinv


def _tc3_body(a0_ref, a1_ref, s2_ref, d0_ref, d1_ref, b2_ref, o_ref):
    dinv = _dinv_block(d0_ref[...], d1_ref[...])
    p2 = (a0_ref[...] + a1_ref[...] + s2_ref[...]) * dinv
    h2 = jax.nn.relu(p2 + b2_ref[...])
    o_ref[...] = h2 * dinv


def _tc4_body(a0_ref, a1_ref, s3_ref, d0_ref, d1_ref, w3_ref, b3_ref, wd1_ref,
              bd1_ref, wd2_ref, bd2_ref, recon_ref, z_ref):
    dinv = _dinv_block(d0_ref[...], d1_ref[...])
    p3 = (a0_ref[...] + a1_ref[...] + s3_ref[...]) * dinv
    z = (
        jnp.dot(p3, w3_ref[...], preferred_element_type=jnp.float32)
        + b3_ref[...]
    )
    h = jax.nn.relu(
        jnp.dot(z, wd1_ref[...], preferred_element_type=jnp.float32)
        + bd1_ref[...]
    )
    recon_ref[...] = (
        jnp.dot(h, wd2_ref[...], preferred_element_type=jnp.float32)
        + bd2_ref[...]
    )
    z_ref[...] = z


def _row_spec(rows, cols):
    return pl.BlockSpec((rows, cols), lambda i: (i, 0))


def _full_spec(shape):
    return pl.BlockSpec(shape, lambda i: tuple(0 for _ in shape))


def kernel(x, edge_index, W1, b1, W2, b2, W3, b3, Wd1, bd1, Wd2, bd2):
    n, d_in = x.shape
    e = edge_index.shape[1]
    h1_w = W1.shape[1]
    h2_w = W2.shape[1]
    lat = W3.shape[1]
    f32 = jnp.float32

    blk = 1024
    n_pad = ((n + blk - 1) // blk) * blk
    edges_per_round = NW * CH * NBUF
    e_pad = ((e + edges_per_round - 1) // edges_per_round) * edges_per_round

    x_p = jnp.pad(x, ((0, n_pad - n), (0, 0)))
    pad_idx = jnp.full((e_pad - e,), n, jnp.int32)
    src_p = jnp.concatenate([edge_index[0], pad_idx])
    dst_p = jnp.concatenate([edge_index[1], pad_idx])
    dst2d = dst_p.reshape(e_pad // CH, CH)
    ones128 = jnp.ones((CH, 128), f32)
    zeros_in = jnp.zeros((n_pad, d_in), f32)
    b1r = b1.reshape(1, -1)
    b2r = b2.reshape(1, -1)
    b3r = b3.reshape(1, -1)
    bd1r = bd1.reshape(1, -1)
    bd2r = bd2.reshape(1, -1)

    grid = (n_pad // blk,)
    hoff = n_pad // blk

    def _half_spec(cols, half):
        return pl.BlockSpec((blk, cols), lambda i, h=half: (i + h * hoff, 0))

    degs = _make_deg(n_pad, e_pad)(dst2d, ones128, zeros_in)

    s1 = pl.pallas_call(
        _tc1_body,
        grid=grid,
        in_specs=[
            _row_spec(blk, d_in),
            _half_spec(128, 0),
            _half_spec(128, 1),
        ],
        out_specs=_row_spec(blk, d_in),
        out_shape=jax.ShapeDtypeStruct((n_pad, d_in), f32),
    )(x_p, degs, degs)

    prop_in = _make_prop(n_pad, e_pad, d_in)

    accs = prop_in(s1, src_p, dst_p, zeros_in)

    s2 = pl.pallas_call(
        _tc2_body,
        grid=grid,
        in_specs=[
            _half_spec(d_in, 0),
            _half_spec(d_in, 1),
            _row_spec(blk, d_in),
            _half_spec(128, 0),
            _half_spec(128, 1),
            _full_spec((d_in, h1_w)),
            _full_spec((1, h1_w)),
            _full_spec((h1_w, h2_w)),
        ],
        out_specs=_row_spec(blk, h2_w),
        out_shape=jax.ShapeDtypeStruct((n_pad, h2_w), f32),
    )(accs, accs, s1, degs, degs, W1, b1r, W2)

    prop_mid = prop_in if h2_w == d_in else _make_prop(n_pad, e_pad, h2_w)
    zeros_mid = zeros_in if h2_w == d_in else jnp.zeros((n_pad, h2_w), f32)
    accs = prop_mid(s2, src_p, dst_p, zeros_mid)

    s3 = pl.pallas_call(
        _tc3_body,
        grid=grid,
        in_specs=[
            _half_spec(h2_w, 0),
            _half_spec(h2_w, 1),
            _row_spec(blk, h2_w),
            _half_spec(128, 0),
            _half_spec(128, 1),
            _full_spec((1, h2_w)),
        ],
        out_specs=_row_spec(blk, h2_w),
        out_shape=jax.ShapeDtypeStruct((n_pad, h2_w), f32),
    )(accs, accs, s2, degs, degs, b2r)

    accs = prop_mid(s3, src_p, dst_p, zeros_mid)

    recon_p, z_p = pl.pallas_call(
        _tc4_body,
        grid=grid,
        in_specs=[
            _half_spec(h2_w, 0),
            _half_spec(h2_w, 1),
            _row_spec(blk, h2_w),
            _half_spec(128, 0),
            _half_spec(128, 1),
            _full_spec((h2_w, lat)),
            _full_spec((1, lat)),
            _full_spec((lat, h2_w)),
            _full_spec((1, h2_w)),
            _full_spec((h2_w, d_in)),
            _full_spec((1, d_in)),
        ],
        out_specs=[
            _row_spec(blk, d_in),
            _row_spec(blk, lat),
        ],
        out_shape=[
            jax.ShapeDtypeStruct((n_pad, d_in), f32),
            jax.ShapeDtypeStruct((n_pad, lat), f32),
        ],
    )(accs, accs, s3, degs, degs, W3, b3r, Wd1, bd1r, Wd2, bd2r)

    return (recon_p[:n], z_p[:n])


# speedup vs baseline: 2.0179x; 2.0179x over previous
import functools

import jax
import jax.numpy as jnp
from jax import lax
from jax.experimental import pallas as pl
from jax.experimental.pallas import tpu as pltpu
from jax.experimental.pallas import tpu_sc as plsc

NC = 2
NS = 16
NW = NC * NS
CH = 128
NBUF = 2


def _mesh():
    return plsc.VectorSubcoreMesh(
        core_axis_name="c", subcore_axis_name="s", num_cores=NC, num_subcores=NS
    )


def _make_deg(n_pad, e_pad):
    epw = e_pad // NW
    n_chunks = epw // CH
    rows_per_sub = n_pad // NS
    grp = 8

    @functools.partial(
        pl.kernel,
        out_type=jax.ShapeDtypeStruct((2 * n_pad, 128), jnp.float32),
        mesh=_mesh(),
        scratch_types=[
            pltpu.VMEM((n_chunks, CH), jnp.int32),
            pltpu.VMEM((CH, 128), jnp.float32),
            pltpu.VMEM_SHARED((n_pad, 128), jnp.float32),
            pltpu.SemaphoreType.DMA,
        ],
    )
    def deg_kernel(dst_hbm, ones_hbm, z_hbm, out_hbm, didx, ones_v, acc, sem):
        c = lax.axis_index("c")
        s = lax.axis_index("s")
        wid = c * NS + s
        r0 = s * rows_per_sub
        row_slc = pl.ds(r0, rows_per_sub)
        pltpu.sync_copy(z_hbm.at[row_slc], acc.at[row_slc])
        pltpu.sync_copy(ones_hbm, ones_v)
        pltpu.sync_copy(dst_hbm.at[pl.ds(wid * n_chunks, n_chunks)], didx)
        plsc.subcore_barrier()

        @pl.loop(0, n_chunks, step=grp)
        def _(g):
            for j in range(grp):
                pltpu.sync_copy(ones_v, acc.at[didx.at[g + j]], add=True)

        plsc.subcore_barrier()
        pltpu.sync_copy(
            acc.at[row_slc], out_hbm.at[pl.ds(c * n_pad + r0, rows_per_sub)]
        )

    return deg_kernel


def _make_prop(n_pad, e_pad, d):
    epw = e_pad // NW
    n_chunks = epw // CH
    rows_per_sub = n_pad // NS

    @functools.partial(
        pl.kernel,
        out_type=jax.ShapeDtypeStruct((2 * n_pad, d), jnp.float32),
        mesh=_mesh(),
        scratch_types=[
            pltpu.VMEM((CH,), jnp.int32),
            pltpu.VMEM((CH,), jnp.int32),
            pltpu.VMEM((CH, d), jnp.float32),
            pltpu.VMEM_SHARED((n_pad, d), jnp.float32),
        ],
    )
    def prop_kernel(s_hbm, src_hbm, dst_hbm, z_hbm, out_hbm,
                    sidx, didx, rows, acc):
        c = lax.axis_index("c")
        s = lax.axis_index("s")
        wid = c * NS + s
        r0 = s * rows_per_sub
        row_slc = pl.ds(r0, rows_per_sub)
        pltpu.sync_copy(z_hbm.at[row_slc], acc.at[row_slc])
        plsc.subcore_barrier()

        base = wid * epw

        @pl.loop(0, n_chunks)
        def _(t):
            off = base + t * CH
            pltpu.sync_copy(src_hbm.at[pl.ds(off, CH)], sidx)
            pltpu.sync_copy(dst_hbm.at[pl.ds(off, CH)], didx)
            pltpu.sync_copy(s_hbm.at[sidx], rows)
            pltpu.sync_copy(rows, acc.at[didx], add=True)

        plsc.subcore_barrier()
        pltpu.sync_copy(
            acc.at[row_slc], out_hbm.at[pl.ds(c * n_pad + r0, rows_per_sub)]
        )

    return prop_kernel


def _dinv_block(d0, d1):
    deg = d0[:, 0:1] + d1[:, 0:1] + 1.0
    return lax.rsqrt(jnp.maximum(deg, 1.0))


def _tc1_body(x_ref, d0_ref, d1_ref, o_ref):
    o_ref[...] = x_ref[...] * _dinv_block(d0_ref[...], d1_ref[...])


def _tc2_body(a0_ref, a1_ref, s1_ref, d0_ref, d1_ref, w1_ref, b1_ref, w2_ref,
              o_ref):
    dinv = _dinv_block(d0_ref[...], d1_ref[...])
    p1 = (a0_ref[...] + a1_ref[...] + s1_ref[...]) * dinv
    h1 = jax.nn.relu(
        jnp.dot(p1, w1_ref[...], preferred_element_type=jnp.float32)
        + b1_ref[...]
    )
    m2 = jnp.dot(h1, w2_ref[...], preferred_element_type=jnp.float32)
    o_ref[...] = m2 * dinv


def _tc3_body(a0_ref, a1_ref, s2_ref, d0_ref, d1_ref, b2_ref, o_ref):
    dinv = _dinv_block(d0_ref[...], d1_ref[...])
    p2 = (a0_ref[...] + a1_ref[...] + s2_ref[...]) * dinv
    h2 = jax.nn.relu(p2 + b2_ref[...])
    o_ref[...] = h2 * dinv


def _tc4_body(a0_ref, a1_ref, s3_ref, d0_ref, d1_ref, w3_ref, b3_ref, wd1_ref,
              bd1_ref, wd2_ref, bd2_ref, recon_ref, z_ref):
    dinv = _dinv_block(d0_ref[...], d1_ref[...])
    p3 = (a0_ref[...] + a1_ref[...] + s3_ref[...]) * dinv
    z = (
        jnp.dot(p3, w3_ref[...], preferred_element_type=jnp.float32)
        + b3_ref[...]
    )
    h = jax.nn.relu(
        jnp.dot(z, wd1_ref[...], preferred_element_type=jnp.float32)
        + bd1_ref[...]
    )
    recon_ref[...] = (
        jnp.dot(h, wd2_ref[...], preferred_element_type=jnp.float32)
        + bd2_ref[...]
    )
    z_ref[...] = z


def _row_spec(rows, cols):
    return pl.BlockSpec((rows, cols), lambda i: (i, 0))


def _full_spec(shape):
    return pl.BlockSpec(shape, lambda i: tuple(0 for _ in shape))


def kernel(x, edge_index, W1, b1, W2, b2, W3, b3, Wd1, bd1, Wd2, bd2):
    n, d_in = x.shape
    e = edge_index.shape[1]
    h1_w = W1.shape[1]
    h2_w = W2.shape[1]
    lat = W3.shape[1]
    f32 = jnp.float32

    blk = 1024
    n_pad = ((n + blk - 1) // blk) * blk
    edges_per_round = NW * CH * NBUF
    e_pad = ((e + edges_per_round - 1) // edges_per_round) * edges_per_round

    x_p = jnp.pad(x, ((0, n_pad - n), (0, 0)))
    pad_idx = (
        jnp.arange(e_pad - e, dtype=jnp.int32) % (n_pad - n) + n
    )
    src_p = jnp.concatenate([edge_index[0], pad_idx])
    dst_p = jnp.concatenate([edge_index[1], pad_idx])
    dst2d = dst_p.reshape(e_pad // CH, CH)
    ones128 = jnp.ones((CH, 128), f32)
    zeros_in = jnp.zeros((n_pad, d_in), f32)
    b1r = b1.reshape(1, -1)
    b2r = b2.reshape(1, -1)
    b3r = b3.reshape(1, -1)
    bd1r = bd1.reshape(1, -1)
    bd2r = bd2.reshape(1, -1)

    grid = (n_pad // blk,)
    hoff = n_pad // blk

    def _half_spec(cols, half):
        return pl.BlockSpec((blk, cols), lambda i, h=half: (i + h * hoff, 0))

    degs = _make_deg(n_pad, e_pad)(dst2d, ones128, zeros_in)

    s1 = pl.pallas_call(
        _tc1_body,
        grid=grid,
        in_specs=[
            _row_spec(blk, d_in),
            _half_spec(128, 0),
            _half_spec(128, 1),
        ],
        out_specs=_row_spec(blk, d_in),
        out_shape=jax.ShapeDtypeStruct((n_pad, d_in), f32),
    )(x_p, degs, degs)

    prop_in = _make_prop(n_pad, e_pad, d_in)

    accs = prop_in(s1, src_p, dst_p, zeros_in)

    s2 = pl.pallas_call(
        _tc2_body,
        grid=grid,
        in_specs=[
            _half_spec(d_in, 0),
            _half_spec(d_in, 1),
            _row_spec(blk, d_in),
            _half_spec(128, 0),
            _half_spec(128, 1),
            _full_spec((d_in, h1_w)),
            _full_spec((1, h1_w)),
            _full_spec((h1_w, h2_w)),
        ],
        out_specs=_row_spec(blk, h2_w),
        out_shape=jax.ShapeDtypeStruct((n_pad, h2_w), f32),
    )(accs, accs, s1, degs, degs, W1, b1r, W2)

    prop_mid = prop_in if h2_w == d_in else _make_prop(n_pad, e_pad, h2_w)
    zeros_mid = zeros_in if h2_w == d_in else jnp.zeros((n_pad, h2_w), f32)
    accs = prop_mid(s2, src_p, dst_p, zeros_mid)

    s3 = pl.pallas_call(
        _tc3_body,
        grid=grid,
        in_specs=[
            _half_spec(h2_w, 0),
            _half_spec(h2_w, 1),
            _row_spec(blk, h2_w),
            _half_spec(128, 0),
            _half_spec(128, 1),
            _full_spec((1, h2_w)),
        ],
        out_specs=_row_spec(blk, h2_w),
        out_shape=jax.ShapeDtypeStruct((n_pad, h2_w), f32),
    )(accs, accs, s2, degs, degs, b2r)

    accs = prop_mid(s3, src_p, dst_p, zeros_mid)

    recon_p, z_p = pl.pallas_call(
        _tc4_body,
        grid=grid,
        in_specs=[
            _half_spec(h2_w, 0),
            _half_spec(h2_w, 1),
            _row_spec(blk, h2_w),
            _half_spec(128, 0),
            _half_spec(128, 1),
            _full_spec((h2_w, lat)),
            _full_spec((1, lat)),
            _full_spec((lat, h2_w)),
            _full_spec((1, h2_w)),
            _full_spec((h2_w, d_in)),
            _full_spec((1, d_in)),
        ],
        out_specs=[
            _row_spec(blk, d_in),
            _row_spec(blk, lat),
        ],
        out_shape=[
            jax.ShapeDtypeStruct((n_pad, d_in), f32),
            jax.ShapeDtypeStruct((n_pad, lat), f32),
        ],
    )(accs, accs, s3, degs, degs, W3, b3r, Wd1, bd1r, Wd2, bd2r)

    return (recon_p[:n], z_p[:n])
